# Initial kernel scaffold; baseline (speedup 1.0000x reference)
import jax
import jax.numpy as jnp
from jax.experimental import pallas as pl


def kernel(x, dw1_w, dw1_b, pw1_w, pw1_b, bn1_g, bn1_b, bn1_m, bn1_v, gc_w, gc_b, dw2_w, dw2_b, pw2_w, pw2_b, bn2_g, bn2_b, bn2_m, bn2_v):
    raise NotImplementedError("write your pallas kernel here")


# speedup vs baseline: 8.8214x; 8.8214x over previous
import functools

import jax
import jax.numpy as jnp
from jax import lax
from jax.experimental import pallas as pl
from jax.experimental.pallas import tpu as pltpu
from jax.experimental.pallas import tpu_sc as plsc

C = 192
C2 = 384
K = 9
N = 1568
NP = 1664
B = 2
BN = B * NP
NW = 32
NODES_PER_W = BN // NW
CHUNK = 8
NCHUNK = NODES_PER_W // CHUNK
ROWS = CHUNK * K

_HI = jax.lax.Precision.HIGHEST
_INF = float("inf")


def _feat_body(x_ref, dw_ref, db_ref, pwT_ref, pwb_ref, m_ref, r_ref,
               g_ref, bb_ref, wa_ref, wb_ref, gcb_ref,
               fn_ref, a_ref, bv_ref):
    xb = x_ref[0]
    h = xb * dw_ref[...] + db_ref[...]
    feat = jnp.dot(h, pwT_ref[...], preferred_element_type=jnp.float32)
    feat = feat + pwb_ref[...]
    feat = (feat - m_ref[...]) / r_ref[...] * g_ref[...] + bb_ref[...]
    nrm = jnp.sqrt(jnp.sum(feat * feat, axis=1, keepdims=True))
    fn = feat / jnp.maximum(nrm, 1e-12)
    fn_ref[0] = fn
    a_ref[0] = jnp.dot(feat, wa_ref[...],
                       preferred_element_type=jnp.float32) + gcb_ref[...]
    bv_ref[0] = jnp.dot(feat, wb_ref[...],
                        preferred_element_type=jnp.float32)


def _run_feat(xTp, dw1, db1, pwT, pwb, bn_m, bn_r, bn_g, bn_b, WA, WB, gcb):
    vec = pl.BlockSpec((1, C), lambda b: (0, 0))
    return pl.pallas_call(
        _feat_body,
        grid=(B,),
        in_specs=[
            pl.BlockSpec((1, NP, C), lambda b: (b, 0, 0)),
            vec, vec,
            pl.BlockSpec((C, C), lambda b: (0, 0)),
            vec, vec, vec, vec, vec,
            pl.BlockSpec((C, C2), lambda b: (0, 0)),
            pl.BlockSpec((C, C2), lambda b: (0, 0)),
            pl.BlockSpec((1, C2), lambda b: (0, 0)),
        ],
        out_specs=[
            pl.BlockSpec((1, NP, C), lambda b: (b, 0, 0)),
            pl.BlockSpec((1, NP, C2), lambda b: (b, 0, 0)),
            pl.BlockSpec((1, NP, C2), lambda b: (b, 0, 0)),
        ],
        out_shape=[
            jax.ShapeDtypeStruct((B, NP, C), jnp.float32),
            jax.ShapeDtypeStruct((B, NP, C2), jnp.float32),
            jax.ShapeDtypeStruct((B, NP, C2), jnp.float32),
        ],
    )(xTp, dw1, db1, pwT, pwb, bn_m, bn_r, bn_g, bn_b, WA, WB, gcb)


_RB = 128
_NRB = NP // _RB


def _knn_body(fnb_ref, fnf_ref, idx_ref):
    fnb = fnb_ref[0]
    fnf = fnf_ref[0]
    sqb = jnp.sum(fnb * fnb, axis=1, keepdims=True)
    sqf = jnp.sum(fnf * fnf, axis=1)
    g = lax.dot_general(fnb, fnf, (((1,), (1,)), ((), ())),
                        preferred_element_type=jnp.float32)
    dist = sqb - 2.0 * g + sqf[None, :]
    iota = lax.broadcasted_iota(jnp.int32, (_RB, NP), 1)
    dist = jnp.where(iota < N, dist, _INF)
    cols = []
    bigi = jnp.int32(1 << 30)
    for _ in range(K):
        m = jnp.min(dist, axis=1, keepdims=True)
        sel = jnp.where(dist == m, iota, bigi)
        j = jnp.min(sel, axis=1, keepdims=True)
        cols.append(j)
        dist = jnp.where(iota == j, _INF, dist)
    idx_ref[0] = jnp.concatenate(cols, axis=1)


def _run_knn(fn):
    return pl.pallas_call(
        _knn_body,
        grid=(B, _NRB),
        in_specs=[
            pl.BlockSpec((1, _RB, C), lambda b, i: (b, i, 0)),
            pl.BlockSpec((1, NP, C), lambda b, i: (b, 0, 0)),
        ],
        out_specs=pl.BlockSpec((1, _RB, K), lambda b, i: (b, i, 0)),
        out_shape=jax.ShapeDtypeStruct((B, NP, K), jnp.int32),
    )(fn, fn)


def _gather_max_body(bv_hbm, gidx_hbm, out_hbm, idx_v, rows_v, out_v, sem):
    wid = lax.axis_index("s") * 2 + lax.axis_index("c")
    node_base = wid * NODES_PER_W

    def chunk_body(ci, carry):
        nb = node_base + ci * CHUNK
        pltpu.sync_copy(gidx_hbm.at[pl.ds(nb * K, ROWS)], idx_v)
        pltpu.async_copy(bv_hbm.at[idx_v], rows_v, sem).wait()

        def node_body(i, c2):
            r0 = i * K
            for j in range(C2 // 16):
                sl = pl.ds(j * 16, 16)
                m = rows_v[r0, sl]
                for k in range(1, K):
                    m = jnp.maximum(m, rows_v[r0 + k, sl])
                out_v[i, sl] = m
            return c2

        lax.fori_loop(0, CHUNK, node_body, 0)
        pltpu.sync_copy(out_v, out_hbm.at[pl.ds(nb, CHUNK)])
        return carry

    lax.fori_loop(0, NCHUNK, chunk_body, 0)


_gather_max = functools.partial(
    pl.kernel,
    out_type=jax.ShapeDtypeStruct((BN, C2), jnp.float32),
    mesh=plsc.VectorSubcoreMesh(core_axis_name="c", subcore_axis_name="s"),
    scratch_types=[
        pltpu.VMEM((ROWS,), jnp.int32),
        pltpu.VMEM((ROWS, C2), jnp.float32),
        pltpu.VMEM((CHUNK, C2), jnp.float32),
        pltpu.SemaphoreType.DMA,
    ],
)(_gather_max_body)


def _fc2_body(a_ref, m_ref, x_ref, m2_ref, b2_ref, out_ref):
    g = jnp.maximum(a_ref[...] + m_ref[...], 0.0)
    out_ref[...] = jnp.dot(g, m2_ref[...],
                           preferred_element_type=jnp.float32) \
        + b2_ref[...] + x_ref[...]


def _run_fc2(A_flat, M_flat, x_flat, M2, b2):
    return pl.pallas_call(
        _fc2_body,
        out_shape=jax.ShapeDtypeStruct((BN, C), jnp.float32),
    )(A_flat, M_flat, x_flat, M2, b2)


def kernel(x, dw1_w, dw1_b, pw1_w, pw1_b, bn1_g, bn1_b, bn1_m, bn1_v,
           gc_w, gc_b, dw2_w, dw2_b, pw2_w, pw2_b, bn2_g, bn2_b, bn2_m,
           bn2_v):
    Bx, Cx, D, H, W = x.shape
    x2 = x.reshape(Bx, Cx, N)
    xT = x2.transpose(0, 2, 1)
    xTp = jnp.pad(xT, ((0, 0), (0, NP - N), (0, 0)))

    WA = (gc_w[:, :C] - gc_w[:, C:]).T
    WB = gc_w[:, C:].T
    gcb = gc_b.reshape(1, C2)

    s2 = bn2_g / jnp.sqrt(bn2_v + 1e-5)
    M2 = (pw2_w * dw2_w[None, :]).T * s2[None, :]
    b2 = (s2 * (pw2_w @ dw2_b + pw2_b - bn2_m) + bn2_b).reshape(1, C)

    r1 = jnp.sqrt(bn1_v + 1e-5)
    fn, A, Bv = _run_feat(
        xTp, dw1_w.reshape(1, C), dw1_b.reshape(1, C), pw1_w.T,
        pw1_b.reshape(1, C), bn1_m.reshape(1, C), r1.reshape(1, C),
        bn1_g.reshape(1, C), bn1_b.reshape(1, C), WA, WB, gcb)
    idx = _run_knn(fn)

    gidx = (idx + (jnp.arange(B, dtype=jnp.int32) * NP)[:, None, None])
    gidx = gidx.reshape(BN * K)
    Mx = _gather_max(Bv.reshape(BN, C2), gidx)

    out_flat = _run_fc2(A.reshape(BN, C2), Mx, xTp.reshape(BN, C), M2, b2)
    out = out_flat.reshape(B, NP, C)[:, :N].transpose(0, 2, 1)
    return out.reshape(Bx, Cx, D, H, W)


# speedup vs baseline: 9.7734x; 1.1079x over previous
import functools

import jax
import jax.numpy as jnp
from jax import lax
from jax.experimental import pallas as pl
from jax.experimental.pallas import tpu as pltpu
from jax.experimental.pallas import tpu_sc as plsc

C = 192
C2 = 384
K = 9
N = 1568
NP = 1664
B = 2
BN = B * NP
NW = 32
NODES_PER_W = BN // NW
CHUNK = 8
NCHUNK = NODES_PER_W // CHUNK
ROWS = CHUNK * K

_RB = 128
_NRB = NP // _RB
_INF = float("inf")


def _feat_knn_body(x_ref, dw_ref, db_ref, pwT_ref, pwb_ref, m_ref, r_ref,
                   g_ref, bb_ref, wa_ref, wb_ref, gcb_ref,
                   a_ref, bv_ref, idx_ref, fn_ref):
    xb = x_ref[0]
    h = xb * dw_ref[...] + db_ref[...]
    feat = jnp.dot(h, pwT_ref[...], preferred_element_type=jnp.float32)
    feat = feat + pwb_ref[...]
    feat = (feat - m_ref[...]) / r_ref[...] * g_ref[...] + bb_ref[...]
    nrm = jnp.sqrt(jnp.sum(feat * feat, axis=1, keepdims=True))
    fn = feat / jnp.maximum(nrm, 1e-12)
    fn_ref[...] = fn
    a_ref[0] = jnp.dot(feat, wa_ref[...],
                       preferred_element_type=jnp.float32) + gcb_ref[...]
    bv_ref[0] = jnp.dot(feat, wb_ref[...],
                        preferred_element_type=jnp.float32)

    sqf = jnp.sum(fn * fn, axis=1)
    bigi = jnp.int32(1 << 30)

    def blk(i, carry):
        fnb = fn_ref[pl.ds(i * _RB, _RB), :]
        sqb = jnp.sum(fnb * fnb, axis=1, keepdims=True)
        g = lax.dot_general(fnb, fn_ref[...], (((1,), (1,)), ((), ())),
                            preferred_element_type=jnp.float32)
        dist = sqb - 2.0 * g + sqf[None, :]
        iota = lax.broadcasted_iota(jnp.int32, (_RB, NP), 1)
        dist = jnp.where(iota < N, dist, _INF)
        cols = []
        for _ in range(K):
            m = jnp.min(dist, axis=1, keepdims=True)
            sel = jnp.where(dist == m, iota, bigi)
            j = jnp.min(sel, axis=1, keepdims=True)
            cols.append(j)
            dist = jnp.where(iota == j, _INF, dist)
        idx_ref[0, pl.ds(i * _RB, _RB), :] = jnp.concatenate(cols, axis=1)
        return carry

    lax.fori_loop(0, _NRB, blk, 0)


def _run_feat_knn(xTp, dw1, db1, pwT, pwb, bn_m, bn_r, bn_g, bn_b, WA, WB,
                  gcb):
    vec = pl.BlockSpec((1, C), lambda b: (0, 0))
    return pl.pallas_call(
        _feat_knn_body,
        grid=(B,),
        in_specs=[
            pl.BlockSpec((1, NP, C), lambda b: (b, 0, 0)),
            vec, vec,
            pl.BlockSpec((C, C), lambda b: (0, 0)),
            vec, vec, vec, vec, vec,
            pl.BlockSpec((C, C2), lambda b: (0, 0)),
            pl.BlockSpec((C, C2), lambda b: (0, 0)),
            pl.BlockSpec((1, C2), lambda b: (0, 0)),
        ],
        out_specs=[
            pl.BlockSpec((1, NP, C2), lambda b: (b, 0, 0)),
            pl.BlockSpec((1, NP, C2), lambda b: (b, 0, 0)),
            pl.BlockSpec((1, NP, K), lambda b: (b, 0, 0)),
        ],
        out_shape=[
            jax.ShapeDtypeStruct((B, NP, C2), jnp.float32),
            jax.ShapeDtypeStruct((B, NP, C2), jnp.float32),
            jax.ShapeDtypeStruct((B, NP, K), jnp.int32),
        ],
        scratch_shapes=[pltpu.VMEM((NP, C), jnp.float32)],
    )(xTp, dw1, db1, pwT, pwb, bn_m, bn_r, bn_g, bn_b, WA, WB, gcb)


def _gather_max_body(bv_hbm, gidx_hbm, out_hbm,
                     idx_v0, idx_v1, rows_v0, rows_v1, out_v, sem0, sem1):
    wid = lax.axis_index("s") * 2 + lax.axis_index("c")
    node_base = wid * NODES_PER_W
    idx_bufs = (idx_v0, idx_v1)
    row_bufs = (rows_v0, rows_v1)
    sems = (sem0, sem1)

    def start(ci, slot):
        nb = node_base + ci * CHUNK
        pltpu.sync_copy(gidx_hbm.at[pl.ds(nb * K, ROWS)], idx_bufs[slot])
        return pltpu.async_copy(bv_hbm.at[idx_bufs[slot]], row_bufs[slot],
                                sems[slot])

    dma = start(0, 0)
    for ci in range(NCHUNK):
        slot = ci % 2
        nxt = dma if ci + 1 >= NCHUNK else start(ci + 1, (ci + 1) % 2)
        dma.wait()
        rows_v = row_bufs[slot]

        def node_body(i, c2, rows_v=rows_v):
            r0 = i * K
            for j in range(C2 // 16):
                sl = pl.ds(j * 16, 16)
                m = rows_v[r0, sl]
                for k in range(1, K):
                    m = jnp.maximum(m, rows_v[r0 + k, sl])
                out_v[i, sl] = m
            return c2

        lax.fori_loop(0, CHUNK, node_body, 0)
        pltpu.sync_copy(out_v, out_hbm.at[pl.ds(node_base + ci * CHUNK,
                                                CHUNK)])
        dma = nxt


_gather_max = functools.partial(
    pl.kernel,
    out_type=jax.ShapeDtypeStruct((BN, C2), jnp.float32),
    mesh=plsc.VectorSubcoreMesh(core_axis_name="c", subcore_axis_name="s"),
    scratch_types=[
        pltpu.VMEM((ROWS,), jnp.int32),
        pltpu.VMEM((ROWS,), jnp.int32),
        pltpu.VMEM((ROWS, C2), jnp.float32),
        pltpu.VMEM((ROWS, C2), jnp.float32),
        pltpu.VMEM((CHUNK, C2), jnp.float32),
        pltpu.SemaphoreType.DMA,
        pltpu.SemaphoreType.DMA,
    ],
)(_gather_max_body)


def _fc2_body(a_ref, m_ref, x_ref, m2_ref, b2_ref, out_ref):
    g = jnp.maximum(a_ref[...] + m_ref[...], 0.0)
    out_ref[...] = jnp.dot(g, m2_ref[...],
                           preferred_element_type=jnp.float32) \
        + b2_ref[...] + x_ref[...]


def _run_fc2(A_flat, M_flat, x_flat, M2, b2):
    return pl.pallas_call(
        _fc2_body,
        out_shape=jax.ShapeDtypeStruct((BN, C), jnp.float32),
    )(A_flat, M_flat, x_flat, M2, b2)


def kernel(x, dw1_w, dw1_b, pw1_w, pw1_b, bn1_g, bn1_b, bn1_m, bn1_v,
           gc_w, gc_b, dw2_w, dw2_b, pw2_w, pw2_b, bn2_g, bn2_b, bn2_m,
           bn2_v):
    Bx, Cx, D, H, W = x.shape
    x2 = x.reshape(Bx, Cx, N)
    xT = x2.transpose(0, 2, 1)
    xTp = jnp.pad(xT, ((0, 0), (0, NP - N), (0, 0)))

    WA = (gc_w[:, :C] - gc_w[:, C:]).T
    WB = gc_w[:, C:].T
    gcb = gc_b.reshape(1, C2)

    s2 = bn2_g / jnp.sqrt(bn2_v + 1e-5)
    M2 = (pw2_w * dw2_w[None, :]).T * s2[None, :]
    b2 = (s2 * (pw2_w @ dw2_b + pw2_b - bn2_m) + bn2_b).reshape(1, C)

    r1 = jnp.sqrt(bn1_v + 1e-5)
    A, Bv, idx = _run_feat_knn(
        xTp, dw1_w.reshape(1, C), dw1_b.reshape(1, C), pw1_w.T,
        pw1_b.reshape(1, C), bn1_m.reshape(1, C), r1.reshape(1, C),
        bn1_g.reshape(1, C), bn1_b.reshape(1, C), WA, WB, gcb)

    gidx = (idx + (jnp.arange(B, dtype=jnp.int32) * NP)[:, None, None])
    gidx = gidx.reshape(BN * K)
    Mx = _gather_max(Bv.reshape(BN, C2), gidx)

    out_flat = _run_fc2(A.reshape(BN, C2), Mx, xTp.reshape(BN, C), M2, b2)
    out = out_flat.reshape(B, NP, C)[:, :N].transpose(0, 2, 1)
    return out.reshape(Bx, Cx, D, H, W)


# speedup vs baseline: 13.2581x; 1.3566x over previous
import functools

import jax
import jax.numpy as jnp
from jax import lax
from jax.experimental import pallas as pl
from jax.experimental.pallas import tpu as pltpu
from jax.experimental.pallas import tpu_sc as plsc

C = 192
C2 = 384
K = 9
N = 1568
NP = 1664
B = 2
NWU = 26
NODES_PER_W = NP // NWU
CHUNK = 8
NCHUNK = NODES_PER_W // CHUNK
ROWS = CHUNK * K

_RB = 416
_NRB = NP // _RB
_INF = float("inf")


def _feat_knn_body(x_ref, dw_ref, db_ref, pwT_ref, pwb_ref, m_ref, r_ref,
                   g_ref, bb_ref, wa_ref, wb_ref, gcb_ref,
                   a_ref, bv_ref, idx_ref, fn_ref):
    xb = x_ref[...]
    h = xb * dw_ref[...] + db_ref[...]
    feat = jnp.dot(h, pwT_ref[...], preferred_element_type=jnp.float32)
    feat = feat + pwb_ref[...]
    feat = (feat - m_ref[...]) / r_ref[...] * g_ref[...] + bb_ref[...]
    nrm = jnp.sqrt(jnp.sum(feat * feat, axis=1, keepdims=True))
    fn = feat / jnp.maximum(nrm, 1e-12)
    fn_ref[...] = fn
    a_ref[...] = jnp.dot(feat, wa_ref[...],
                         preferred_element_type=jnp.float32) + gcb_ref[...]
    bv_ref[...] = jnp.dot(feat, wb_ref[...],
                          preferred_element_type=jnp.float32)

    sqf = jnp.sum(fn * fn, axis=1)

    def blk(i, carry):
        fnb = fn_ref[pl.ds(i * _RB, _RB), :]
        sqb = jnp.sum(fnb * fnb, axis=1, keepdims=True)
        g = lax.dot_general(fnb, fn_ref[...], (((1,), (1,)), ((), ())),
                            preferred_element_type=jnp.float32)
        dist = sqb - 2.0 * g + sqf[None, :]
        iota = lax.broadcasted_iota(jnp.int32, (_RB, NP), 1)
        dist = jnp.where(iota < N, dist, _INF)
        cols = []
        for _ in range(K):
            j = jnp.argmin(dist, axis=1)[:, None].astype(jnp.int32)
            cols.append(j)
            dist = jnp.where(iota == j, _INF, dist)
        idx_ref[pl.ds(i * _RB, _RB), :] = jnp.concatenate(cols, axis=1)
        return carry

    lax.fori_loop(0, _NRB, blk, 0)


def _run_feat_knn(xb, dw1, db1, pwT, pwb, bn_m, bn_r, bn_g, bn_b, WA, WB,
                  gcb):
    return pl.pallas_call(
        _feat_knn_body,
        out_shape=[
            jax.ShapeDtypeStruct((NP, C2), jnp.float32),
            jax.ShapeDtypeStruct((NP, C2), jnp.float32),
            jax.ShapeDtypeStruct((NP, K), jnp.int32),
        ],
        scratch_shapes=[pltpu.VMEM((NP, C), jnp.float32)],
    )(xb, dw1, db1, pwT, pwb, bn_m, bn_r, bn_g, bn_b, WA, WB, gcb)


def _gather_max_body(bv_hbm, gidx_hbm, out_hbm,
                     idx_v0, idx_v1, rows_v0, rows_v1, out_v, sem0, sem1):
    wid = lax.axis_index("s") * 2 + lax.axis_index("c")

    @pl.when(wid < NWU)
    def _():
        node_base = wid * NODES_PER_W
        idx_bufs = (idx_v0, idx_v1)
        row_bufs = (rows_v0, rows_v1)
        sems = (sem0, sem1)

        def start(ci, slot):
            nb = node_base + ci * CHUNK
            pltpu.sync_copy(gidx_hbm.at[pl.ds(nb * K, ROWS)], idx_bufs[slot])
            return pltpu.async_copy(bv_hbm.at[idx_bufs[slot]],
                                    row_bufs[slot], sems[slot])

        dma = start(0, 0)
        for ci in range(NCHUNK):
            slot = ci % 2
            nxt = dma if ci + 1 >= NCHUNK else start(ci + 1, (ci + 1) % 2)
            dma.wait()
            rows_v = row_bufs[slot]

            def node_body(i, c2, rows_v=rows_v):
                r0 = i * K
                for j in range(C2 // 16):
                    sl = pl.ds(j * 16, 16)
                    m = rows_v[r0, sl]
                    for k in range(1, K):
                        m = jnp.maximum(m, rows_v[r0 + k, sl])
                    out_v[i, sl] = m
                return c2

            lax.fori_loop(0, CHUNK, node_body, 0)
            pltpu.sync_copy(out_v, out_hbm.at[pl.ds(node_base + ci * CHUNK,
                                                    CHUNK)])
            dma = nxt


_gather_max = functools.partial(
    pl.kernel,
    out_type=jax.ShapeDtypeStruct((NP, C2), jnp.float32),
    mesh=plsc.VectorSubcoreMesh(core_axis_name="c", subcore_axis_name="s"),
    scratch_types=[
        pltpu.VMEM((ROWS,), jnp.int32),
        pltpu.VMEM((ROWS,), jnp.int32),
        pltpu.VMEM((ROWS, C2), jnp.float32),
        pltpu.VMEM((ROWS, C2), jnp.float32),
        pltpu.VMEM((CHUNK, C2), jnp.float32),
        pltpu.SemaphoreType.DMA,
        pltpu.SemaphoreType.DMA,
    ],
)(_gather_max_body)


def _fc2_body(a_ref, m_ref, x_ref, m2_ref, b2_ref, out_ref):
    g = jnp.maximum(a_ref[...] + m_ref[...], 0.0)
    outT = lax.dot_general(m2_ref[...], g, (((0,), (1,)), ((), ())),
                           preferred_element_type=jnp.float32)
    out_ref[...] = outT[:, :N] + b2_ref[...] + x_ref[...]


def _run_fc2(A2, M2x, x2b, M2, b2):
    return pl.pallas_call(
        _fc2_body,
        out_shape=jax.ShapeDtypeStruct((C, N), jnp.float32),
    )(A2, M2x, x2b, M2, b2)


def kernel(x, dw1_w, dw1_b, pw1_w, pw1_b, bn1_g, bn1_b, bn1_m, bn1_v,
           gc_w, gc_b, dw2_w, dw2_b, pw2_w, pw2_b, bn2_g, bn2_b, bn2_m,
           bn2_v):
    Bx, Cx, D, H, W = x.shape
    x2 = x.reshape(Bx, Cx, N)
    xT = x2.transpose(0, 2, 1)
    xTp = jnp.pad(xT, ((0, 0), (0, NP - N), (0, 0)))

    WA = (gc_w[:, :C] - gc_w[:, C:]).T
    WB = gc_w[:, C:].T
    gcb = gc_b.reshape(1, C2)

    s2 = bn2_g / jnp.sqrt(bn2_v + 1e-5)
    M2 = (pw2_w * dw2_w[None, :]).T * s2[None, :]
    b2 = (s2 * (pw2_w @ dw2_b + pw2_b - bn2_m) + bn2_b).reshape(C, 1)

    r1 = jnp.sqrt(bn1_v + 1e-5)
    fc1_args = (dw1_w.reshape(1, C), dw1_b.reshape(1, C), pw1_w.T,
                pw1_b.reshape(1, C), bn1_m.reshape(1, C), r1.reshape(1, C),
                bn1_g.reshape(1, C), bn1_b.reshape(1, C), WA, WB, gcb)

    outs = []
    abi = [_run_feat_knn(xTp[b], *fc1_args) for b in range(B)]
    mxs = [_gather_max(A_Bv_idx[1], A_Bv_idx[2].reshape(NP * K))
           for A_Bv_idx in abi]
    for b in range(B):
        outs.append(_run_fc2(abi[b][0], mxs[b], x2[b], M2, b2))
    return jnp.stack(outs).reshape(Bx, Cx, D, H, W)


# speedup vs baseline: 13.6335x; 1.0283x over previous
import functools

import jax
import jax.numpy as jnp
from jax import lax
from jax.experimental import pallas as pl
from jax.experimental.pallas import tpu as pltpu
from jax.experimental.pallas import tpu_sc as plsc

C = 192
C2 = 384
K = 9
N = 1568
NP = 1664
B = 2
NWU = 26
NODES_PER_W = NP // NWU
CHUNK = 8
NCHUNK = NODES_PER_W // CHUNK
ROWS = CHUNK * K

_RB = 416
_NRB = NP // _RB
_INF = float("inf")


def _feat_knn_body(x_ref, dw_ref, db_ref, pw_ref, pwb_ref, m_ref, r_ref,
                   g_ref, bb_ref, wa_ref, wb_ref, gcb_ref,
                   a_ref, bv_ref, idx_ref, fn_ref):
    xb = x_ref[...]
    h = xb * dw_ref[...] + db_ref[...]
    feat = lax.dot_general(h, pw_ref[...], (((1,), (1,)), ((), ())),
                           preferred_element_type=jnp.float32)
    feat = feat + pwb_ref[...]
    feat = (feat - m_ref[...]) / r_ref[...] * g_ref[...] + bb_ref[...]
    nrm = jnp.sqrt(jnp.sum(feat * feat, axis=1, keepdims=True))
    fn = feat / jnp.maximum(nrm, 1e-12)
    fn_ref[...] = fn
    a_ref[...] = lax.dot_general(
        feat, wa_ref[...], (((1,), (1,)), ((), ())),
        preferred_element_type=jnp.float32) + gcb_ref[...]
    bv_ref[...] = lax.dot_general(
        feat, wb_ref[...], (((1,), (1,)), ((), ())),
        preferred_element_type=jnp.float32)

    sqf = jnp.sum(fn * fn, axis=1)

    def blk(i, carry):
        fnb = fn_ref[pl.ds(i * _RB, _RB), :]
        sqb = jnp.sum(fnb * fnb, axis=1, keepdims=True)
        g = lax.dot_general(fnb, fn_ref[...], (((1,), (1,)), ((), ())),
                            preferred_element_type=jnp.float32)
        dist = sqb - 2.0 * g + sqf[None, :]
        iota = lax.broadcasted_iota(jnp.int32, (_RB, NP), 1)
        dist = jnp.where(iota < N, dist, _INF)
        cols = []
        for _ in range(K):
            j = jnp.argmin(dist, axis=1)[:, None].astype(jnp.int32)
            cols.append(j)
            dist = jnp.where(iota == j, _INF, dist)
        idx_ref[pl.ds(i * _RB, _RB), :] = jnp.concatenate(cols, axis=1)
        return carry

    lax.fori_loop(0, _NRB, blk, 0)


def _run_feat_knn(xb, dw1, db1, pwT, pwb, bn_m, bn_r, bn_g, bn_b, WA, WB,
                  gcb):
    return pl.pallas_call(
        _feat_knn_body,
        out_shape=[
            jax.ShapeDtypeStruct((NP, C2), jnp.float32),
            jax.ShapeDtypeStruct((NP, C2), jnp.float32),
            jax.ShapeDtypeStruct((NP, K), jnp.int32),
        ],
        scratch_shapes=[pltpu.VMEM((NP, C), jnp.float32)],
    )(xb, dw1, db1, pwT, pwb, bn_m, bn_r, bn_g, bn_b, WA, WB, gcb)


def _gather_max_body(bv_hbm, gidx_hbm, out_hbm,
                     idx_v0, idx_v1, rows_v0, rows_v1, out_v, sem0, sem1):
    wid = lax.axis_index("s") * 2 + lax.axis_index("c")

    @pl.when(wid < NWU)
    def _():
        node_base = wid * NODES_PER_W
        idx_bufs = (idx_v0, idx_v1)
        row_bufs = (rows_v0, rows_v1)
        sems = (sem0, sem1)

        def start(ci, slot):
            nb = node_base + ci * CHUNK
            pltpu.sync_copy(gidx_hbm.at[pl.ds(nb * K, ROWS)], idx_bufs[slot])
            return pltpu.async_copy(bv_hbm.at[idx_bufs[slot]],
                                    row_bufs[slot], sems[slot])

        dma = start(0, 0)
        for ci in range(NCHUNK):
            slot = ci % 2
            nxt = dma if ci + 1 >= NCHUNK else start(ci + 1, (ci + 1) % 2)
            dma.wait()
            rows_v = row_bufs[slot]

            def node_body(i, c2, rows_v=rows_v):
                r0 = i * K
                for j in range(C2 // 16):
                    sl = pl.ds(j * 16, 16)
                    m = rows_v[r0, sl]
                    for k in range(1, K):
                        m = jnp.maximum(m, rows_v[r0 + k, sl])
                    out_v[i, sl] = m
                return c2

            lax.fori_loop(0, CHUNK, node_body, 0)
            pltpu.sync_copy(out_v, out_hbm.at[pl.ds(node_base + ci * CHUNK,
                                                    CHUNK)])
            dma = nxt


_gather_max = functools.partial(
    pl.kernel,
    out_type=jax.ShapeDtypeStruct((NP, C2), jnp.float32),
    mesh=plsc.VectorSubcoreMesh(core_axis_name="c", subcore_axis_name="s"),
    scratch_types=[
        pltpu.VMEM((ROWS,), jnp.int32),
        pltpu.VMEM((ROWS,), jnp.int32),
        pltpu.VMEM((ROWS, C2), jnp.float32),
        pltpu.VMEM((ROWS, C2), jnp.float32),
        pltpu.VMEM((CHUNK, C2), jnp.float32),
        pltpu.SemaphoreType.DMA,
        pltpu.SemaphoreType.DMA,
    ],
)(_gather_max_body)


def _fc2_body(a_ref, m_ref, x_ref, m2_ref, b2_ref, out_ref):
    g = jnp.maximum(a_ref[...] + m_ref[...], 0.0)
    outT = lax.dot_general(m2_ref[...], g, (((1,), (1,)), ((), ())),
                           preferred_element_type=jnp.float32)
    out_ref[...] = outT[:, :N] + b2_ref[...] + x_ref[...]


def _run_fc2(A2, M2x, x2b, M2, b2):
    return pl.pallas_call(
        _fc2_body,
        out_shape=jax.ShapeDtypeStruct((C, N), jnp.float32),
    )(A2, M2x, x2b, M2, b2)


def kernel(x, dw1_w, dw1_b, pw1_w, pw1_b, bn1_g, bn1_b, bn1_m, bn1_v,
           gc_w, gc_b, dw2_w, dw2_b, pw2_w, pw2_b, bn2_g, bn2_b, bn2_m,
           bn2_v):
    Bx, Cx, D, H, W = x.shape
    x2 = x.reshape(Bx, Cx, N)
    xT = x2.transpose(0, 2, 1)
    xTp = jnp.pad(xT, ((0, 0), (0, NP - N), (0, 0)))

    WA = gc_w[:, :C] - gc_w[:, C:]
    WB = gc_w[:, C:]
    gcb = gc_b.reshape(1, C2)

    s2 = bn2_g / jnp.sqrt(bn2_v + 1e-5)
    M2 = pw2_w * dw2_w[None, :] * s2[:, None]
    b2 = (s2 * (pw2_w @ dw2_b + pw2_b - bn2_m) + bn2_b).reshape(C, 1)

    r1 = jnp.sqrt(bn1_v + 1e-5)
    fc1_args = (dw1_w.reshape(1, C), dw1_b.reshape(1, C), pw1_w,
                pw1_b.reshape(1, C), bn1_m.reshape(1, C), r1.reshape(1, C),
                bn1_g.reshape(1, C), bn1_b.reshape(1, C), WA, WB, gcb)

    outs = []
    abi = [_run_feat_knn(xTp[b], *fc1_args) for b in range(B)]
    mxs = [_gather_max(A_Bv_idx[1], A_Bv_idx[2].reshape(NP * K))
           for A_Bv_idx in abi]
    for b in range(B):
        outs.append(_run_fc2(abi[b][0], mxs[b], x2[b], M2, b2))
    return jnp.stack(outs).reshape(Bx, Cx, D, H, W)


# speedup vs baseline: 14.4450x; 1.0595x over previous
import functools

import jax
import jax.numpy as jnp
from jax import lax
from jax.experimental import pallas as pl
from jax.experimental.pallas import tpu as pltpu
from jax.experimental.pallas import tpu_sc as plsc

C = 192
C2 = 384
K = 9
N = 1568
NP = 1664
B = 2
NWU = 26
NODES_PER_W = NP // NWU
CHUNK = 8
NCHUNK = NODES_PER_W // CHUNK
ROWS = CHUNK * K

_RB = 416
_NRB = NP // _RB
_INF = float("inf")


def _feat_knn_body(x_ref, dw_ref, db_ref, pw_ref, pwb_ref, m_ref, r_ref,
                   g_ref, bb_ref, wa_ref, wb_ref, gcb_ref,
                   a_ref, bv_ref, idx_ref, fn_ref):
    xb = x_ref[...]
    h = xb * dw_ref[...] + db_ref[...]
    feat = lax.dot_general(h, pw_ref[...], (((1,), (1,)), ((), ())),
                           preferred_element_type=jnp.float32)
    feat = feat + pwb_ref[...]
    feat = (feat - m_ref[...]) / r_ref[...] * g_ref[...] + bb_ref[...]
    nrm = jnp.sqrt(jnp.sum(feat * feat, axis=1, keepdims=True))
    fn = feat / jnp.maximum(nrm, 1e-12)
    fn_ref[...] = fn
    a_ref[...] = lax.dot_general(
        feat, wa_ref[...], (((1,), (1,)), ((), ())),
        preferred_element_type=jnp.float32) + gcb_ref[...]
    bv_ref[...] = lax.dot_general(
        feat, wb_ref[...], (((1,), (1,)), ((), ())),
        preferred_element_type=jnp.float32)

    sqf = jnp.sum(fn * fn, axis=1)

    def blk(i, carry):
        fnb = fn_ref[pl.ds(i * _RB, _RB), :]
        sqb = jnp.sum(fnb * fnb, axis=1, keepdims=True)
        g = lax.dot_general(fnb, fn_ref[...], (((1,), (1,)), ((), ())),
                            preferred_element_type=jnp.float32)
        dist = sqb - 2.0 * g + sqf[None, :]
        iota = lax.broadcasted_iota(jnp.int32, (_RB, NP), 1)
        dist = jnp.where(iota < N, dist, _INF)
        cols = []
        for _ in range(K):
            j = jnp.argmin(dist, axis=1)[:, None].astype(jnp.int32)
            cols.append(j)
            dist = jnp.where(iota == j, _INF, dist)
        idx_ref[pl.ds(i * _RB, _RB), :] = jnp.concatenate(cols, axis=1)
        return carry

    lax.fori_loop(0, _NRB, blk, 0)


def _run_feat_knn(xb, dw1, db1, pwT, pwb, bn_m, bn_r, bn_g, bn_b, WA, WB,
                  gcb):
    return pl.pallas_call(
        _feat_knn_body,
        out_shape=[
            jax.ShapeDtypeStruct((NP, C2), jnp.float32),
            jax.ShapeDtypeStruct((NP, C2), jnp.float32),
            jax.ShapeDtypeStruct((NP, K), jnp.int32),
        ],
        scratch_shapes=[pltpu.VMEM((NP, C), jnp.float32)],
    )(xb, dw1, db1, pwT, pwb, bn_m, bn_r, bn_g, bn_b, WA, WB, gcb)


def _gather_max_body(bv_hbm, gidx_hbm, out_hbm,
                     idx_v0, idx_v1, rows_v0, rows_v1, out_v, sem0, sem1):
    wid = lax.axis_index("s") * 2 + lax.axis_index("c")

    @pl.when(wid < NWU)
    def _():
        node_base = wid * NODES_PER_W
        idx_bufs = (idx_v0, idx_v1)
        row_bufs = (rows_v0, rows_v1)
        sems = (sem0, sem1)

        def start(ci, slot):
            nb = node_base + ci * CHUNK
            pltpu.sync_copy(gidx_hbm.at[pl.ds(nb * K, ROWS)], idx_bufs[slot])
            return pltpu.async_copy(bv_hbm.at[idx_bufs[slot]],
                                    row_bufs[slot], sems[slot])

        dma = start(0, 0)
        for ci in range(NCHUNK):
            slot = ci % 2
            nxt = dma if ci + 1 >= NCHUNK else start(ci + 1, (ci + 1) % 2)
            dma.wait()
            rows_v = row_bufs[slot]

            def col_body(jj, c2, rows_v=rows_v):
                sl = pl.ds(jj * 16, 16)
                for i in range(CHUNK):
                    r0 = i * K
                    m = rows_v[r0, sl]
                    for k in range(1, K):
                        m = jnp.maximum(m, rows_v[r0 + k, sl])
                    out_v[i, sl] = m
                return c2

            lax.fori_loop(0, C2 // 16, col_body, 0)
            pltpu.sync_copy(out_v, out_hbm.at[pl.ds(node_base + ci * CHUNK,
                                                    CHUNK)])
            dma = nxt


_gather_max = functools.partial(
    pl.kernel,
    out_type=jax.ShapeDtypeStruct((NP, C2), jnp.float32),
    mesh=plsc.VectorSubcoreMesh(core_axis_name="c", subcore_axis_name="s"),
    scratch_types=[
        pltpu.VMEM((ROWS,), jnp.int32),
        pltpu.VMEM((ROWS,), jnp.int32),
        pltpu.VMEM((ROWS, C2), jnp.float32),
        pltpu.VMEM((ROWS, C2), jnp.float32),
        pltpu.VMEM((CHUNK, C2), jnp.float32),
        pltpu.SemaphoreType.DMA,
        pltpu.SemaphoreType.DMA,
    ],
)(_gather_max_body)


def _fc2_body(a_ref, m_ref, x_ref, m2_ref, b2_ref, out_ref):
    g = jnp.maximum(a_ref[...] + m_ref[...], 0.0)
    outT = lax.dot_general(m2_ref[...], g, (((1,), (1,)), ((), ())),
                           preferred_element_type=jnp.float32)
    out_ref[...] = outT[:, :N] + b2_ref[...] + x_ref[...]


def _run_fc2(A2, M2x, x2b, M2, b2):
    return pl.pallas_call(
        _fc2_body,
        out_shape=jax.ShapeDtypeStruct((C, N), jnp.float32),
    )(A2, M2x, x2b, M2, b2)


def kernel(x, dw1_w, dw1_b, pw1_w, pw1_b, bn1_g, bn1_b, bn1_m, bn1_v,
           gc_w, gc_b, dw2_w, dw2_b, pw2_w, pw2_b, bn2_g, bn2_b, bn2_m,
           bn2_v):
    Bx, Cx, D, H, W = x.shape
    x2 = x.reshape(Bx, Cx, N)
    xT = x2.transpose(0, 2, 1)
    xTp = jnp.pad(xT, ((0, 0), (0, NP - N), (0, 0)))

    WA = gc_w[:, :C] - gc_w[:, C:]
    WB = gc_w[:, C:]
    gcb = gc_b.reshape(1, C2)

    s2 = bn2_g / jnp.sqrt(bn2_v + 1e-5)
    M2 = pw2_w * dw2_w[None, :] * s2[:, None]
    b2 = (s2 * (pw2_w @ dw2_b + pw2_b - bn2_m) + bn2_b).reshape(C, 1)

    r1 = jnp.sqrt(bn1_v + 1e-5)
    fc1_args = (dw1_w.reshape(1, C), dw1_b.reshape(1, C), pw1_w,
                pw1_b.reshape(1, C), bn1_m.reshape(1, C), r1.reshape(1, C),
                bn1_g.reshape(1, C), bn1_b.reshape(1, C), WA, WB, gcb)

    outs = []
    abi = [_run_feat_knn(xTp[b], *fc1_args) for b in range(B)]
    mxs = [_gather_max(A_Bv_idx[1], A_Bv_idx[2].reshape(NP * K))
           for A_Bv_idx in abi]
    for b in range(B):
        outs.append(_run_fc2(abi[b][0], mxs[b], x2[b], M2, b2))
    return jnp.stack(outs).reshape(Bx, Cx, D, H, W)


# speedup vs baseline: 14.4495x; 1.0003x over previous
import functools

import jax
import jax.numpy as jnp
from jax import lax
from jax.experimental import pallas as pl
from jax.experimental.pallas import tpu as pltpu
from jax.experimental.pallas import tpu_sc as plsc

C = 192
C2 = 384
K = 9
N = 1568
B = 2
NWU = 28
NODES_PER_W = N // NWU
CHUNK = 8
NCHUNK = NODES_PER_W // CHUNK
ROWS = CHUNK * K

_RB = 392
_NRB = N // _RB
_INF = float("inf")


def _feat_knn_body(x_ref, dw_ref, db_ref, pw_ref, pwb_ref, m_ref, r_ref,
                   g_ref, bb_ref, gcw_ref, gcb_ref,
                   a_ref, bv_ref, idx_ref, fn_ref):
    h = x_ref[...] * dw_ref[...] + db_ref[...]
    feat = lax.dot_general(h, pw_ref[...], (((0,), (1,)), ((), ())),
                           preferred_element_type=jnp.float32)
    feat = feat + pwb_ref[...]
    feat = (feat - m_ref[...]) / r_ref[...] * g_ref[...] + bb_ref[...]
    nrm = jnp.sqrt(jnp.sum(feat * feat, axis=1, keepdims=True))
    fn = feat / jnp.maximum(nrm, 1e-12)
    fn_ref[...] = fn
    gcw = gcw_ref[...]
    wa = gcw[:, :C] - gcw[:, C:]
    a_ref[...] = lax.dot_general(
        feat, wa, (((1,), (1,)), ((), ())),
        preferred_element_type=jnp.float32) + gcb_ref[...]
    bv_ref[...] = lax.dot_general(
        feat, gcw[:, C:], (((1,), (1,)), ((), ())),
        preferred_element_type=jnp.float32)

    sqf = jnp.sum(fn * fn, axis=1)

    def blk(i, carry):
        fnb = fn_ref[pl.ds(i * _RB, _RB), :]
        sqb = jnp.sum(fnb * fnb, axis=1, keepdims=True)
        g = lax.dot_general(fnb, fn_ref[...], (((1,), (1,)), ((), ())),
                            preferred_element_type=jnp.float32)
        dist = sqb - 2.0 * g + sqf[None, :]
        iota = lax.broadcasted_iota(jnp.int32, (_RB, N), 1)
        cols = []
        for _ in range(K):
            j = jnp.argmin(dist, axis=1)[:, None].astype(jnp.int32)
            cols.append(j)
            dist = jnp.where(iota == j, _INF, dist)
        idx_ref[pl.ds(i * _RB, _RB), :] = jnp.concatenate(cols, axis=1)
        return carry

    lax.fori_loop(0, _NRB, blk, 0)


def _run_feat_knn(x2b, dw1, db1, pw1, pwb, bn_m, bn_r, bn_g, bn_b, gcw,
                  gcb):
    return pl.pallas_call(
        _feat_knn_body,
        out_shape=[
            jax.ShapeDtypeStruct((N, C2), jnp.float32),
            jax.ShapeDtypeStruct((N, C2), jnp.float32),
            jax.ShapeDtypeStruct((N, K), jnp.int32),
        ],
        scratch_shapes=[pltpu.VMEM((N, C), jnp.float32)],
    )(x2b, dw1, db1, pw1, pwb, bn_m, bn_r, bn_g, bn_b, gcw, gcb)


def _gather_max_body(bv_hbm, gidx_hbm, out_hbm,
                     idx_v0, idx_v1, rows_v0, rows_v1, out_v, sem0, sem1):
    wid = lax.axis_index("s") * 2 + lax.axis_index("c")

    @pl.when(wid < NWU)
    def _():
        node_base = wid * NODES_PER_W
        idx_bufs = (idx_v0, idx_v1)
        row_bufs = (rows_v0, rows_v1)
        sems = (sem0, sem1)

        def start(ci, slot):
            nb = node_base + ci * CHUNK
            pltpu.sync_copy(gidx_hbm.at[pl.ds(nb * K, ROWS)], idx_bufs[slot])
            return pltpu.async_copy(bv_hbm.at[idx_bufs[slot]],
                                    row_bufs[slot], sems[slot])

        dma = start(0, 0)
        for ci in range(NCHUNK):
            slot = ci % 2
            nxt = dma if ci + 1 >= NCHUNK else start(ci + 1, (ci + 1) % 2)
            dma.wait()
            rows_v = row_bufs[slot]

            def col_body(jj, c2, rows_v=rows_v):
                sl = pl.ds(jj * 16, 16)
                for i in range(CHUNK):
                    r0 = i * K
                    m = rows_v[r0, sl]
                    for k in range(1, K):
                        m = jnp.maximum(m, rows_v[r0 + k, sl])
                    out_v[i, sl] = m
                return c2

            lax.fori_loop(0, C2 // 16, col_body, 0)
            pltpu.sync_copy(out_v, out_hbm.at[pl.ds(node_base + ci * CHUNK,
                                                    CHUNK)])
            dma = nxt


_gather_max = functools.partial(
    pl.kernel,
    out_type=jax.ShapeDtypeStruct((N, C2), jnp.float32),
    mesh=plsc.VectorSubcoreMesh(core_axis_name="c", subcore_axis_name="s"),
    scratch_types=[
        pltpu.VMEM((ROWS,), jnp.int32),
        pltpu.VMEM((ROWS,), jnp.int32),
        pltpu.VMEM((ROWS, C2), jnp.float32),
        pltpu.VMEM((ROWS, C2), jnp.float32),
        pltpu.VMEM((CHUNK, C2), jnp.float32),
        pltpu.SemaphoreType.DMA,
        pltpu.SemaphoreType.DMA,
    ],
)(_gather_max_body)


def _fc2_body(a_ref, m_ref, x_ref, pw2_ref, dw2_ref, s2_ref, b2_ref,
              out_ref):
    g = jnp.maximum(a_ref[...] + m_ref[...], 0.0)
    m2 = pw2_ref[...] * dw2_ref[...] * s2_ref[...]
    outT = lax.dot_general(m2, g, (((1,), (1,)), ((), ())),
                           preferred_element_type=jnp.float32)
    out_ref[...] = outT + b2_ref[...] + x_ref[...]


def _run_fc2(A2, M2x, x2b, pw2, dw2, s2, b2):
    return pl.pallas_call(
        _fc2_body,
        out_shape=jax.ShapeDtypeStruct((C, N), jnp.float32),
    )(A2, M2x, x2b, pw2, dw2, s2, b2)


def kernel(x, dw1_w, dw1_b, pw1_w, pw1_b, bn1_g, bn1_b, bn1_m, bn1_v,
           gc_w, gc_b, dw2_w, dw2_b, pw2_w, pw2_b, bn2_g, bn2_b, bn2_m,
           bn2_v):
    Bx, Cx, D, H, W = x.shape
    x2 = x.reshape(Bx, Cx, N)

    gcb = gc_b.reshape(1, C2)
    s2 = bn2_g / jnp.sqrt(bn2_v + 1e-5)
    b2 = (s2 * (pw2_w @ dw2_b + pw2_b - bn2_m) + bn2_b).reshape(C, 1)

    r1 = jnp.sqrt(bn1_v + 1e-5)
    fc1_args = (dw1_w.reshape(C, 1), dw1_b.reshape(C, 1), pw1_w,
                pw1_b.reshape(1, C), bn1_m.reshape(1, C), r1.reshape(1, C),
                bn1_g.reshape(1, C), bn1_b.reshape(1, C), gc_w, gcb)

    abi = [_run_feat_knn(x2[b], *fc1_args) for b in range(B)]
    mxs = [_gather_max(t[1], t[2].reshape(N * K)) for t in abi]
    outs = [_run_fc2(abi[b][0], mxs[b], x2[b], pw2_w,
                     dw2_w.reshape(1, C2), s2.reshape(C, 1), b2)
            for b in range(B)]
    return jnp.stack(outs).reshape(Bx, Cx, D, H, W)


# speedup vs baseline: 15.2684x; 1.0567x over previous
import functools

import jax
import jax.numpy as jnp
from jax import lax
from jax.experimental import pallas as pl
from jax.experimental.pallas import tpu as pltpu
from jax.experimental.pallas import tpu_sc as plsc

C = 192
C2 = 384
K = 9
N = 1568
B = 2
NWU = 28
NODES_PER_W = N // NWU
CHUNK = 8
NCHUNK = NODES_PER_W // CHUNK
ROWS = CHUNK * K

_RB = 392
_NRB = N // _RB
_INF = float("inf")


def _feat_knn_body(x_ref, dw_ref, db_ref, pw_ref, pwb_ref, m_ref, r_ref,
                   g_ref, bb_ref, gcw_ref, gcb_ref,
                   a_ref, bv_ref, idx_ref, fn_ref):
    h = x_ref[0] * dw_ref[...] + db_ref[...]
    feat = lax.dot_general(h, pw_ref[...], (((0,), (1,)), ((), ())),
                           preferred_element_type=jnp.float32)
    feat = feat + pwb_ref[...]
    feat = (feat - m_ref[...]) / r_ref[...] * g_ref[...] + bb_ref[...]
    nrm = jnp.sqrt(jnp.sum(feat * feat, axis=1, keepdims=True))
    fn = feat / jnp.maximum(nrm, 1e-12)
    fn_ref[...] = fn
    gcw = gcw_ref[...]
    wa = gcw[:, :C] - gcw[:, C:]
    a_ref[...] = lax.dot_general(
        feat, wa, (((1,), (1,)), ((), ())),
        preferred_element_type=jnp.float32) + gcb_ref[...]
    bv_ref[...] = lax.dot_general(
        feat, gcw[:, C:], (((1,), (1,)), ((), ())),
        preferred_element_type=jnp.float32)

    sqf = jnp.sum(fn * fn, axis=1)

    def blk(i, carry):
        fnb = fn_ref[pl.ds(i * _RB, _RB), :]
        sqb = jnp.sum(fnb * fnb, axis=1, keepdims=True)
        g = lax.dot_general(fnb, fn_ref[...], (((1,), (1,)), ((), ())),
                            preferred_element_type=jnp.float32)
        dist = sqb - 2.0 * g + sqf[None, :]
        iota = lax.broadcasted_iota(jnp.int32, (_RB, N), 1)
        cols = []
        for _ in range(K):
            j = jnp.argmin(dist, axis=1)[:, None].astype(jnp.int32)
            cols.append(j)
            dist = jnp.where(iota == j, _INF, dist)
        idx_ref[pl.ds(i * _RB, _RB), :] = jnp.concatenate(cols, axis=1)
        return carry

    lax.fori_loop(0, _NRB, blk, 0)


def _run_feat_knn(bsel, x2, dw1, db1, pw1, pwb, bn_m, bn_r, bn_g, bn_b,
                  gcw, gcb):
    full = lambda s: pl.BlockSpec(s, lambda _: tuple(0 for _ in s))
    return pl.pallas_call(
        _feat_knn_body,
        grid=(1,),
        in_specs=[
            pl.BlockSpec((1, C, N), lambda _: (bsel, 0, 0)),
            full((C, 1)), full((C, 1)), full((C, C)), full((1, C)),
            full((1, C)), full((1, C)), full((1, C)), full((1, C)),
            full((C2, C2)), full((1, C2)),
        ],
        out_specs=[full((N, C2)), full((N, C2)), full((N, K))],
        out_shape=[
            jax.ShapeDtypeStruct((N, C2), jnp.float32),
            jax.ShapeDtypeStruct((N, C2), jnp.float32),
            jax.ShapeDtypeStruct((N, K), jnp.int32),
        ],
        scratch_shapes=[pltpu.VMEM((N, C), jnp.float32)],
    )(x2, dw1, db1, pw1, pwb, bn_m, bn_r, bn_g, bn_b, gcw, gcb)


def _gather_max_body(bv_hbm, gidx_hbm, out_hbm,
                     idx_v0, idx_v1, rows_v0, rows_v1, out_v, sem0, sem1):
    wid = lax.axis_index("s") * 2 + lax.axis_index("c")

    @pl.when(wid < NWU)
    def _():
        node_base = wid * NODES_PER_W
        idx_bufs = (idx_v0, idx_v1)
        row_bufs = (rows_v0, rows_v1)
        sems = (sem0, sem1)

        def start(ci, slot):
            nb = node_base + ci * CHUNK
            pltpu.sync_copy(gidx_hbm.at[pl.ds(nb * K, ROWS)], idx_bufs[slot])
            return pltpu.async_copy(bv_hbm.at[idx_bufs[slot]],
                                    row_bufs[slot], sems[slot])

        dma = start(0, 0)
        for ci in range(NCHUNK):
            slot = ci % 2
            nxt = dma if ci + 1 >= NCHUNK else start(ci + 1, (ci + 1) % 2)
            dma.wait()
            rows_v = row_bufs[slot]

            def col_body(jj, c2, rows_v=rows_v):
                sl = pl.ds(jj * 16, 16)
                for i in range(CHUNK):
                    r0 = i * K
                    m = rows_v[r0, sl]
                    for k in range(1, K):
                        m = jnp.maximum(m, rows_v[r0 + k, sl])
                    out_v[i, sl] = m
                return c2

            lax.fori_loop(0, C2 // 16, col_body, 0)
            pltpu.sync_copy(out_v, out_hbm.at[pl.ds(node_base + ci * CHUNK,
                                                    CHUNK)])
            dma = nxt


_gather_max = functools.partial(
    pl.kernel,
    out_type=jax.ShapeDtypeStruct((N, C2), jnp.float32),
    mesh=plsc.VectorSubcoreMesh(core_axis_name="c", subcore_axis_name="s"),
    scratch_types=[
        pltpu.VMEM((ROWS,), jnp.int32),
        pltpu.VMEM((ROWS,), jnp.int32),
        pltpu.VMEM((ROWS, C2), jnp.float32),
        pltpu.VMEM((ROWS, C2), jnp.float32),
        pltpu.VMEM((CHUNK, C2), jnp.float32),
        pltpu.SemaphoreType.DMA,
        pltpu.SemaphoreType.DMA,
    ],
)(_gather_max_body)


def _fc2_body(a_ref, m_ref, x_ref, pw2_ref, dw2_ref, s2_ref, b2_ref,
              out_ref):
    g = jnp.maximum(a_ref[...] + m_ref[...], 0.0)
    m2 = pw2_ref[...] * dw2_ref[...] * s2_ref[...]
    outT = lax.dot_general(m2, g, (((1,), (1,)), ((), ())),
                           preferred_element_type=jnp.float32)
    out_ref[...] = outT + b2_ref[...] + x_ref[0]


def _run_fc2(bsel, A2, M2x, x2, pw2, dw2, s2, b2):
    full = lambda s: pl.BlockSpec(s, lambda _: tuple(0 for _ in s))
    return pl.pallas_call(
        _fc2_body,
        grid=(1,),
        in_specs=[
            full((N, C2)), full((N, C2)),
            pl.BlockSpec((1, C, N), lambda _: (bsel, 0, 0)),
            full((C, C2)), full((1, C2)), full((C, 1)), full((C, 1)),
        ],
        out_specs=full((C, N)),
        out_shape=jax.ShapeDtypeStruct((C, N), jnp.float32),
    )(A2, M2x, x2, pw2, dw2, s2, b2)


def kernel(x, dw1_w, dw1_b, pw1_w, pw1_b, bn1_g, bn1_b, bn1_m, bn1_v,
           gc_w, gc_b, dw2_w, dw2_b, pw2_w, pw2_b, bn2_g, bn2_b, bn2_m,
           bn2_v):
    Bx, Cx, D, H, W = x.shape
    x2 = x.reshape(Bx, Cx, N)

    gcb = gc_b.reshape(1, C2)
    s2 = bn2_g / jnp.sqrt(bn2_v + 1e-5)
    b2 = (s2 * (pw2_w @ dw2_b + pw2_b - bn2_m) + bn2_b).reshape(C, 1)

    r1 = jnp.sqrt(bn1_v + 1e-5)
    fc1_args = (dw1_w.reshape(C, 1), dw1_b.reshape(C, 1), pw1_w,
                pw1_b.reshape(1, C), bn1_m.reshape(1, C), r1.reshape(1, C),
                bn1_g.reshape(1, C), bn1_b.reshape(1, C), gc_w, gcb)

    abi = [_run_feat_knn(b, x2, *fc1_args) for b in range(B)]
    mxs = [_gather_max(t[1], t[2].reshape(N * K)) for t in abi]
    outs = [_run_fc2(b, abi[b][0], mxs[b], x2, pw2_w,
                     dw2_w.reshape(1, C2), s2.reshape(C, 1), b2)
            for b in range(B)]
    return jnp.stack(outs).reshape(Bx, Cx, D, H, W)
